# baseline (speedup 1.0000x reference)
import functools
import jax
import jax.numpy as jnp
from jax import lax
from jax.experimental import pallas as pl
from jax.experimental.pallas import tpu as pltpu
from jax.experimental.pallas import tpu_sc as plsc

NUM_EDGES = 3_200_000
NUM_BUS = 100_000
ROW = 128
NROWS = NUM_EDGES // ROW
NCORES = 2
NSUB = 16
NW = NCORES * NSUB
ROWS_BIG = 784
N_BIG = 21
CROWS = 32
NFULL = 24
TROWS = 8
NB_PAD = 100_352
ZSLICE = NB_PAD // NSUB
ZB = 1568


def _sc_body(idx_hbm, p_hbm, q_hbm, pp_hbm, qp_hbm,
             idx_a, pa, qa, idx_b, pb_, qb_, idx_ta, pta, qta,
             idx_tb, ptb, qtb, zb, acc_p, acc_q, in_sem, sc_sem):
    c = lax.axis_index("c")
    s = lax.axis_index("s")
    w = c * NSUB + s
    r0w = w * ROWS_BIG - jnp.maximum(w - N_BIG, 0) * TROWS

    big = [(idx_a, pa, qa), (idx_b, pb_, qb_)]
    small = [(idx_ta, pta, qta), (idx_tb, ptb, qtb)]
    steps = [(r0w + i * CROWS, CROWS, False, big[i % 2])
             for i in range(NFULL)]
    steps.append((r0w + NFULL * CROWS, TROWS, False, small[0]))
    steps.append((r0w + NFULL * CROWS + TROWS, TROWS, True, small[1]))

    def in_copies(bufs, r0, n):
        iv, pb, qb = bufs
        return (
            pltpu.make_async_copy(idx_hbm.at[pl.ds(2 * r0, 2 * n)], iv,
                                  in_sem),
            pltpu.make_async_copy(p_hbm.at[pl.ds(r0, n)], pb, in_sem),
            pltpu.make_async_copy(q_hbm.at[pl.ds(r0, n)], qb, in_sem),
        )

    def start_inputs(bufs, r0, n):
        for cp in in_copies(bufs, r0, n):
            cp.start()

    def wait_inputs(bufs, r0, n):
        for cp in in_copies(bufs, r0, n):
            cp.wait()

    def fire(bufs, n):
        iv, pb, qb = bufs

        def body(i, _):
            pltpu.async_copy(pb.at[i], acc_p.at[iv.at[2 * i]], sc_sem,
                             add=True)
            pltpu.async_copy(qb.at[i], acc_q.at[iv.at[2 * i]], sc_sem,
                             add=True)
            return 0
        lax.fori_loop(0, n, body, 0)

    def drain_scatters(bufs, n):
        iv, pb, qb = bufs

        def body(i, _):
            pltpu.make_async_copy(pb.at[0], acc_p.at[iv.at[0]],
                                  sc_sem).wait()
            return 0
        lax.fori_loop(0, 2 * n, body, 0)

    def guarded(cond, fn):
        if cond:
            pl.when(w < N_BIG)(fn)
        else:
            fn()

    start_inputs(steps[0][3], steps[0][0], steps[0][1])

    def zstore(i, _):
        zb[pl.ds(i * 16, 16)] = jnp.zeros((16,), jnp.float32)
        return 0
    lax.fori_loop(0, ZB // 16, zstore, 0)
    base = s * ZSLICE
    for k in range(ZSLICE // ZB):
        pltpu.sync_copy(zb, acc_p.at[pl.ds(base + k * ZB, ZB)])
        pltpu.sync_copy(zb, acc_q.at[pl.ds(base + k * ZB, ZB)])
    plsc.subcore_barrier()

    for k, (r0, n, cond, bufs) in enumerate(steps):
        guarded(cond,
                lambda bufs=bufs, r0=r0, n=n: wait_inputs(bufs, r0, n))
        if k >= 1:
            drain_scatters(steps[k - 1][3], steps[k - 1][1])
        if k + 1 < len(steps):
            nr0, nn, ncond, nbufs = steps[k + 1]
            guarded(ncond,
                    lambda nbufs=nbufs, nr0=nr0, nn=nn:
                    start_inputs(nbufs, nr0, nn))
        guarded(cond, lambda bufs=bufs, n=n: fire(bufs, n))
    guarded(True, lambda: drain_scatters(steps[-1][3], steps[-1][1]))

    plsc.subcore_barrier()

    pltpu.sync_copy(acc_p.at[pl.ds(s * ZSLICE, ZSLICE)],
                    pp_hbm.at[c, pl.ds(s * ZSLICE, ZSLICE)])
    pltpu.sync_copy(acc_q.at[pl.ds(s * ZSLICE, ZSLICE)],
                    qp_hbm.at[c, pl.ds(s * ZSLICE, ZSLICE)])


_sc_scatter = functools.partial(
    pl.kernel,
    out_type=(jax.ShapeDtypeStruct((NCORES, NB_PAD), jnp.float32),
              jax.ShapeDtypeStruct((NCORES, NB_PAD), jnp.float32)),
    mesh=plsc.VectorSubcoreMesh(core_axis_name="c", subcore_axis_name="s"),
    scratch_types=[
        pltpu.VMEM((2 * CROWS, ROW), jnp.int32),
        pltpu.VMEM((CROWS, ROW), jnp.float32),
        pltpu.VMEM((CROWS, ROW), jnp.float32),
        pltpu.VMEM((2 * CROWS, ROW), jnp.int32),
        pltpu.VMEM((CROWS, ROW), jnp.float32),
        pltpu.VMEM((CROWS, ROW), jnp.float32),
        pltpu.VMEM((2 * TROWS, ROW), jnp.int32),
        pltpu.VMEM((TROWS, ROW), jnp.float32),
        pltpu.VMEM((TROWS, ROW), jnp.float32),
        pltpu.VMEM((2 * TROWS, ROW), jnp.int32),
        pltpu.VMEM((TROWS, ROW), jnp.float32),
        pltpu.VMEM((TROWS, ROW), jnp.float32),
        pltpu.VMEM((ZB,), jnp.float32),
        pltpu.VMEM_SHARED((NB_PAD,), jnp.float32),
        pltpu.VMEM_SHARED((NB_PAD,), jnp.float32),
        pltpu.SemaphoreType.DMA,
        pltpu.SemaphoreType.DMA,
    ],
)(_sc_body)


def _combine_body(pp_ref, qp_ref, po_ref, qo_ref):
    po_ref[...] = pp_ref[0, :] + pp_ref[1, :]
    qo_ref[...] = qp_ref[0, :] + qp_ref[1, :]


_CB = 14_336

_combine = pl.pallas_call(
    _combine_body,
    grid=(NB_PAD // _CB,),
    in_specs=[pl.BlockSpec((NCORES, _CB), lambda i: (0, i)),
              pl.BlockSpec((NCORES, _CB), lambda i: (0, i))],
    out_specs=[pl.BlockSpec((_CB,), lambda i: (i,)),
               pl.BlockSpec((_CB,), lambda i: (i,))],
    out_shape=(jax.ShapeDtypeStruct((NB_PAD,), jnp.float32),
               jax.ShapeDtypeStruct((NB_PAD,), jnp.float32)),
)


def kernel(Pft, Qft, edge_index, num_bus):
    idx = edge_index.astype(jnp.int32).reshape(2, NROWS, ROW)
    idx = idx.transpose(1, 0, 2).reshape(2 * NROWS, ROW)
    p2 = Pft.reshape(NROWS, ROW)
    q2 = Qft.reshape(NROWS, ROW)
    pp, qp = _sc_scatter(idx, p2, q2)
    P, Q = _combine(pp, qp)
    return P[:NUM_BUS], Q[:NUM_BUS]


# speedup vs baseline: 1.0116x; 1.0116x over previous
import functools
import jax
import jax.numpy as jnp
from jax import lax
from jax.experimental import pallas as pl
from jax.experimental.pallas import tpu as pltpu
from jax.experimental.pallas import tpu_sc as plsc

NUM_EDGES = 3_200_000
NUM_BUS = 100_000
ROW = 128
NROWS = NUM_EDGES // ROW
NCORES = 2
NSUB = 16
NW = NCORES * NSUB
ROWS_BIG = 784
N_BIG = 21
CROWS = 64
NFULL = 12
TROWS = 8
NB_PAD = 100_352
ZSLICE = NB_PAD // NSUB
ZB = 1568


def _sc_body(idx_hbm, p_hbm, q_hbm, pp_hbm, qp_hbm,
             idx_a, pa, qa, idx_b, pb_, qb_, idx_ta, pta, qta,
             idx_tb, ptb, qtb, zb, acc_p, acc_q, in_sem, sc_sem):
    c = lax.axis_index("c")
    s = lax.axis_index("s")
    w = c * NSUB + s
    r0w = w * ROWS_BIG - jnp.maximum(w - N_BIG, 0) * TROWS

    big = [(idx_a, pa, qa), (idx_b, pb_, qb_)]
    small = [(idx_ta, pta, qta), (idx_tb, ptb, qtb)]
    steps = [(r0w + i * CROWS, CROWS, False, big[i % 2])
             for i in range(NFULL)]
    steps.append((r0w + NFULL * CROWS, TROWS, False, small[0]))
    steps.append((r0w + NFULL * CROWS + TROWS, TROWS, True, small[1]))

    def in_copies(bufs, r0, n):
        iv, pb, qb = bufs
        return (
            pltpu.make_async_copy(idx_hbm.at[pl.ds(2 * r0, 2 * n)], iv,
                                  in_sem),
            pltpu.make_async_copy(p_hbm.at[pl.ds(r0, n)], pb, in_sem),
            pltpu.make_async_copy(q_hbm.at[pl.ds(r0, n)], qb, in_sem),
        )

    def start_inputs(bufs, r0, n):
        for cp in in_copies(bufs, r0, n):
            cp.start()

    def wait_inputs(bufs, r0, n):
        for cp in in_copies(bufs, r0, n):
            cp.wait()

    def fire(bufs, n):
        iv, pb, qb = bufs

        def body(i, _):
            pltpu.async_copy(pb.at[i], acc_p.at[iv.at[2 * i]], sc_sem,
                             add=True)
            pltpu.async_copy(qb.at[i], acc_q.at[iv.at[2 * i]], sc_sem,
                             add=True)
            return 0
        lax.fori_loop(0, n, body, 0)

    def drain_scatters(bufs, n):
        iv, pb, qb = bufs

        def body(i, _):
            pltpu.make_async_copy(pb.at[0], acc_p.at[iv.at[0]],
                                  sc_sem).wait()
            return 0
        lax.fori_loop(0, 2 * n, body, 0)

    def guarded(cond, fn):
        if cond:
            pl.when(w < N_BIG)(fn)
        else:
            fn()

    start_inputs(steps[0][3], steps[0][0], steps[0][1])

    def zstore(i, _):
        zb[pl.ds(i * 16, 16)] = jnp.zeros((16,), jnp.float32)
        return 0
    lax.fori_loop(0, ZB // 16, zstore, 0)
    base = s * ZSLICE
    for k in range(ZSLICE // ZB):
        pltpu.sync_copy(zb, acc_p.at[pl.ds(base + k * ZB, ZB)])
        pltpu.sync_copy(zb, acc_q.at[pl.ds(base + k * ZB, ZB)])
    plsc.subcore_barrier()

    for k, (r0, n, cond, bufs) in enumerate(steps):
        guarded(cond,
                lambda bufs=bufs, r0=r0, n=n: wait_inputs(bufs, r0, n))
        if k >= 1:
            drain_scatters(steps[k - 1][3], steps[k - 1][1])
        if k + 1 < len(steps):
            nr0, nn, ncond, nbufs = steps[k + 1]
            guarded(ncond,
                    lambda nbufs=nbufs, nr0=nr0, nn=nn:
                    start_inputs(nbufs, nr0, nn))
        guarded(cond, lambda bufs=bufs, n=n: fire(bufs, n))
    guarded(True, lambda: drain_scatters(steps[-1][3], steps[-1][1]))

    plsc.subcore_barrier()

    pltpu.sync_copy(acc_p.at[pl.ds(s * ZSLICE, ZSLICE)],
                    pp_hbm.at[c, pl.ds(s * ZSLICE, ZSLICE)])
    pltpu.sync_copy(acc_q.at[pl.ds(s * ZSLICE, ZSLICE)],
                    qp_hbm.at[c, pl.ds(s * ZSLICE, ZSLICE)])


_sc_scatter = functools.partial(
    pl.kernel,
    out_type=(jax.ShapeDtypeStruct((NCORES, NB_PAD), jnp.float32),
              jax.ShapeDtypeStruct((NCORES, NB_PAD), jnp.float32)),
    mesh=plsc.VectorSubcoreMesh(core_axis_name="c", subcore_axis_name="s"),
    scratch_types=[
        pltpu.VMEM((2 * CROWS, ROW), jnp.int32),
        pltpu.VMEM((CROWS, ROW), jnp.float32),
        pltpu.VMEM((CROWS, ROW), jnp.float32),
        pltpu.VMEM((2 * CROWS, ROW), jnp.int32),
        pltpu.VMEM((CROWS, ROW), jnp.float32),
        pltpu.VMEM((CROWS, ROW), jnp.float32),
        pltpu.VMEM((2 * TROWS, ROW), jnp.int32),
        pltpu.VMEM((TROWS, ROW), jnp.float32),
        pltpu.VMEM((TROWS, ROW), jnp.float32),
        pltpu.VMEM((2 * TROWS, ROW), jnp.int32),
        pltpu.VMEM((TROWS, ROW), jnp.float32),
        pltpu.VMEM((TROWS, ROW), jnp.float32),
        pltpu.VMEM((ZB,), jnp.float32),
        pltpu.VMEM_SHARED((NB_PAD,), jnp.float32),
        pltpu.VMEM_SHARED((NB_PAD,), jnp.float32),
        pltpu.SemaphoreType.DMA,
        pltpu.SemaphoreType.DMA,
    ],
)(_sc_body)


def _combine_body(pp_ref, qp_ref, po_ref, qo_ref):
    po_ref[...] = pp_ref[0, :] + pp_ref[1, :]
    qo_ref[...] = qp_ref[0, :] + qp_ref[1, :]


_CB = 14_336

_combine = pl.pallas_call(
    _combine_body,
    grid=(NB_PAD // _CB,),
    in_specs=[pl.BlockSpec((NCORES, _CB), lambda i: (0, i)),
              pl.BlockSpec((NCORES, _CB), lambda i: (0, i))],
    out_specs=[pl.BlockSpec((_CB,), lambda i: (i,)),
               pl.BlockSpec((_CB,), lambda i: (i,))],
    out_shape=(jax.ShapeDtypeStruct((NB_PAD,), jnp.float32),
               jax.ShapeDtypeStruct((NB_PAD,), jnp.float32)),
)


def kernel(Pft, Qft, edge_index, num_bus):
    idx = edge_index.astype(jnp.int32).reshape(2, NROWS, ROW)
    idx = idx.transpose(1, 0, 2).reshape(2 * NROWS, ROW)
    p2 = Pft.reshape(NROWS, ROW)
    q2 = Qft.reshape(NROWS, ROW)
    pp, qp = _sc_scatter(idx, p2, q2)
    P, Q = _combine(pp, qp)
    return P[:NUM_BUS], Q[:NUM_BUS]
